# Initial kernel scaffold; baseline (speedup 1.0000x reference)
import jax
import jax.numpy as jnp
from jax.experimental import pallas as pl


def kernel(coord, feat, offset, segment, params):
    raise NotImplementedError("write your pallas kernel here")


# speedup vs baseline: 3.3622x; 3.3622x over previous
import functools
import jax
import jax.numpy as jnp
from jax import lax
from jax.experimental import pallas as pl
from jax.experimental.pallas import tpu as pltpu

N_NODES = 10000
HID = 80
K = 7
N_LAYERS = 4
BIG = 3.0e38

MM_R = 256
KN_R = 128


def _pad_to(n, m):
    return ((n + m - 1) // m) * m


def _knn_call(crow, ccol, n, k, exclude_diag):
    Np = crow.shape[0]
    Np2 = ccol.shape[1]
    R = KN_R

    def body(crow_ref, ccol_ref, out_ref):
        i = pl.program_id(0)
        cr = crow_ref[...]
        xi = cr[:, 0:1]
        yi = cr[:, 1:2]
        zi = cr[:, 2:3]
        cx = ccol_ref[0:1, :]
        cy = ccol_ref[1:2, :]
        cz = ccol_ref[2:3, :]
        sqi = xi * xi + yi * yi + zi * zi
        sqj = cx * cx + cy * cy + cz * cz
        cross = jnp.dot(cr.astype(jnp.bfloat16),
                        ccol_ref[...].astype(jnp.bfloat16),
                        preferred_element_type=jnp.float32)
        d = sqi + sqj - 2.0 * cross
        col = lax.broadcasted_iota(jnp.int32, (R, Np2), 1)
        d = jnp.where(col >= n, BIG, d)
        if exclude_diag:
            rowg = lax.broadcasted_iota(jnp.int32, (R, Np2), 0) + i * R
            d = jnp.where(col == rowg, BIG, d)
        for kk in range(k):
            m = jnp.min(d, axis=1, keepdims=True)
            idx = jnp.min(jnp.where(d <= m, col, Np2), axis=1, keepdims=True)
            out_ref[:, kk:kk + 1] = idx
            if kk + 1 < k:
                d = jnp.where(col == idx, BIG, d)

    return pl.pallas_call(
        body,
        grid=(Np // R,),
        in_specs=[
            pl.BlockSpec((R, 8), lambda i: (i, 0)),
            pl.BlockSpec((8, Np2), lambda i: (0, 0)),
        ],
        out_specs=pl.BlockSpec((R, 8), lambda i: (i, 0)),
        out_shape=jax.ShapeDtypeStruct((Np, 8), jnp.int32),
    )(crow, ccol)


def _bn_mm_call(x, g, m, r, b, w):
    Np, Din = x.shape
    Dout = w.shape[1]
    R = MM_R

    def body(x_ref, g_ref, m_ref, r_ref, b_ref, w_ref, t_ref):
        z = (g_ref[0:1, :] * (x_ref[...] - m_ref[0:1, :])) * r_ref[0:1, :] \
            + b_ref[0:1, :]
        t_ref[...] = jnp.dot(z, w_ref[...], preferred_element_type=jnp.float32)

    row = lambda v: v.reshape(1, -1)
    return pl.pallas_call(
        body,
        grid=(Np // R,),
        in_specs=[
            pl.BlockSpec((R, Din), lambda i: (i, 0)),
            pl.BlockSpec((1, Din), lambda i: (0, 0)),
            pl.BlockSpec((1, Din), lambda i: (0, 0)),
            pl.BlockSpec((1, Din), lambda i: (0, 0)),
            pl.BlockSpec((1, Din), lambda i: (0, 0)),
            pl.BlockSpec((Din, Dout), lambda i: (0, 0)),
        ],
        out_specs=pl.BlockSpec((R, Dout), lambda i: (i, 0)),
        out_shape=jax.ShapeDtypeStruct((Np, Dout), jnp.float32),
    )(x, row(g), row(m), row(r), row(b), w)


def _bn_relu_call(t, g, m, r, b):
    Np, D = t.shape
    R = MM_R

    def body(t_ref, g_ref, m_ref, r_ref, b_ref, o_ref):
        z = (g_ref[0:1, :] * (t_ref[...] - m_ref[0:1, :])) * r_ref[0:1, :] \
            + b_ref[0:1, :]
        o_ref[...] = jnp.maximum(z, 0.0)

    row = lambda v: v.reshape(1, -1)
    return pl.pallas_call(
        body,
        grid=(Np // R,),
        in_specs=[pl.BlockSpec((R, D), lambda i: (i, 0))] +
                 [pl.BlockSpec((1, D), lambda i: (0, 0))] * 4,
        out_specs=pl.BlockSpec((R, D), lambda i: (i, 0)),
        out_shape=jax.ShapeDtypeStruct((Np, D), jnp.float32),
    )(t, row(g), row(m), row(r), row(b))


def _mm_call(x, w):
    Np, Din = x.shape
    Dout = w.shape[1]
    R = MM_R

    def body(x_ref, w_ref, o_ref):
        o_ref[...] = jnp.dot(x_ref[...], w_ref[...],
                             preferred_element_type=jnp.float32)

    return pl.pallas_call(
        body,
        grid=(Np // R,),
        in_specs=[
            pl.BlockSpec((R, Din), lambda i: (i, 0)),
            pl.BlockSpec((Din, Dout), lambda i: (0, 0)),
        ],
        out_specs=pl.BlockSpec((R, Dout), lambda i: (i, 0)),
        out_shape=jax.ShapeDtypeStruct((Np, Dout), jnp.float32),
    )(x, w)


def _y_call(h, msum, w_self):
    Np, D = h.shape
    R = MM_R

    def body(h_ref, m_ref, w_ref, y_ref):
        y = jnp.dot(h_ref[...], w_ref[...], preferred_element_type=jnp.float32)
        y_ref[...] = y + m_ref[...] / 7.0

    return pl.pallas_call(
        body,
        grid=(Np // R,),
        in_specs=[
            pl.BlockSpec((R, D), lambda i: (i, 0)),
            pl.BlockSpec((R, D), lambda i: (i, 0)),
            pl.BlockSpec((D, D), lambda i: (0, 0)),
        ],
        out_specs=pl.BlockSpec((R, D), lambda i: (i, 0)),
        out_shape=jax.ShapeDtypeStruct((Np, D), jnp.float32),
    )(h, msum, w_self)


def _post_call(y, g, m, r, b, h):
    Np, D = y.shape
    R = MM_R

    def body(y_ref, g_ref, m_ref, r_ref, b_ref, h_ref, h2_ref):
        z = (g_ref[0:1, :] * (y_ref[...] - m_ref[0:1, :])) * r_ref[0:1, :] \
            + b_ref[0:1, :]
        z = jnp.where(z > 0, z, jnp.exp(jnp.minimum(z, 0.0)) - 1.0)
        h2_ref[...] = z + h_ref[...]

    row = lambda v: v.reshape(1, -1)
    return pl.pallas_call(
        body,
        grid=(Np // R,),
        in_specs=[pl.BlockSpec((R, D), lambda i: (i, 0))] +
                 [pl.BlockSpec((1, D), lambda i: (0, 0))] * 4 +
                 [pl.BlockSpec((R, D), lambda i: (i, 0))],
        out_specs=pl.BlockSpec((R, D), lambda i: (i, 0)),
        out_shape=jax.ShapeDtypeStruct((Np, D), jnp.float32),
    )(y, row(g), row(m), row(r), row(b), h)


def _head_call(x, w0, b0, w1, b1, w2, b2):
    Np, D = x.shape
    D1 = w0.shape[1]
    R = MM_R

    def body(x_ref, w0_ref, b0_ref, w1_ref, b1_ref, w2_ref, b2_ref, y_ref):
        def stage(v, w_r, b_r):
            z = jnp.dot(v, w_r[...], preferred_element_type=jnp.float32)
            z = z + b_r[0:1, :]
            return jnp.where(z > 0, z, jnp.exp(jnp.minimum(z, 0.0)) - 1.0)

        y = stage(x_ref[...], w0_ref, b0_ref)
        y = stage(y, w1_ref, b1_ref)
        y_ref[...] = stage(y, w2_ref, b2_ref)

    return pl.pallas_call(
        body,
        grid=(Np // R,),
        in_specs=[
            pl.BlockSpec((R, D), lambda i: (i, 0)),
            pl.BlockSpec((D, D1), lambda i: (0, 0)),
            pl.BlockSpec((1, D1), lambda i: (0, 0)),
            pl.BlockSpec((D1, D1), lambda i: (0, 0)),
            pl.BlockSpec((1, D1), lambda i: (0, 0)),
            pl.BlockSpec((D1, D1), lambda i: (0, 0)),
            pl.BlockSpec((1, D1), lambda i: (0, 0)),
        ],
        out_specs=pl.BlockSpec((R, D1), lambda i: (i, 0)),
        out_shape=jax.ShapeDtypeStruct((Np, D1), jnp.float32),
    )(x, w0, b0.reshape(1, -1), w1, b1.reshape(1, -1), w2, b2.reshape(1, -1))


def _final_call(y, g, m, r, b, w):
    Np, D = y.shape
    Dout = w.shape[1]
    R = MM_R

    def body(y_ref, g_ref, m_ref, r_ref, b_ref, w_ref, o_ref):
        z = (g_ref[0:1, :] * (y_ref[...] - m_ref[0:1, :])) * r_ref[0:1, :] \
            + b_ref[0:1, :]
        o_ref[...] = jnp.dot(z, w_ref[...], preferred_element_type=jnp.float32)

    row = lambda v: v.reshape(1, -1)
    return pl.pallas_call(
        body,
        grid=(Np // R,),
        in_specs=[pl.BlockSpec((R, D), lambda i: (i, 0))] +
                 [pl.BlockSpec((1, D), lambda i: (0, 0))] * 4 +
                 [pl.BlockSpec((D, Dout), lambda i: (0, 0))],
        out_specs=pl.BlockSpec((R, Dout), lambda i: (i, 0)),
        out_shape=jax.ShapeDtypeStruct((Np, Dout), jnp.float32),
    )(y, row(g), row(m), row(r), row(b), w)


def _gather_sum7(msg, nbr_flat, n_nodes_pad):
    g = jnp.take(msg, nbr_flat, axis=0)
    return g.reshape(n_nodes_pad, K, HID).sum(axis=1)


def _gather_rows2(t1, t2, idx):
    return jnp.take(t1, idx, axis=0), jnp.take(t2, idx, axis=0)


def _compose_final(feats3, up3, l3, l2, l1, l0):
    t = jnp.take(up3, l3)
    t = jnp.take(t, l2)
    t = jnp.take(t, l1)
    t = jnp.take(t, l0)
    return jnp.take(feats3, t, axis=0)


def _stats(x):
    m = jnp.mean(x, axis=0)
    v = jnp.var(x, axis=0)
    return m, lax.rsqrt(v + 1e-5)


def kernel(coord, feat, offset, segment, params):
    n0 = coord.shape[0]
    Np0 = _pad_to(n0, 1024)

    crow = jnp.zeros((Np0, 8), jnp.float32).at[:n0, :3].set(coord)
    ccol = crow.T
    feat8 = jnp.zeros((Np0, 8), jnp.float32).at[:n0, :6].set(feat)

    ones8 = jnp.ones((8,), jnp.float32)
    zeros8 = jnp.zeros((8,), jnp.float32)

    m1, r1 = _stats(feat8[:n0])
    w_emb8 = jnp.zeros((8, HID), jnp.float32).at[:6, :].set(params['W_emb'])
    t = _bn_mm_call(feat8, ones8, m1, r1, zeros8, w_emb8)
    m2, r2 = _stats(t[:n0])
    h = _bn_relu_call(t, params['bne_g'], m2, r2, params['bne_b'])

    n = n0
    Np = Np0
    locs = []
    ups = []
    for l in range(N_LAYERS):
        lp = params['layers'][l]
        nbr = _knn_call(crow, ccol, n, K, True)
        msg = _mm_call(h, lp['W_msg'])
        nbr_flat = nbr[:, :K].reshape(-1)
        msum = _gather_sum7(msg, nbr_flat, Np)
        y = _y_call(h, msum, lp['W_self'])
        my, ry = _stats(y[:n])
        h2 = _post_call(y, lp['bn_g'], my, ry, lp['bn_b'], h)
        score = h2[:n] @ lp['score']
        n_up = n // 2
        Mp = Np // 2
        _, up = lax.top_k(score, n_up)
        up_pad = jnp.zeros((Mp,), jnp.int32).at[:n_up].set(up)

        h_next, crow_up = _gather_rows2(h2, crow, up_pad)
        ccol_up = crow_up.T
        local = _knn_call(crow, ccol_up, n_up, 1, False)[:, 0]
        locs.append(local)
        ups.append(up_pad)

        if l == N_LAYERS - 1:
            feats3 = h2
        h = h_next
        crow = crow_up
        ccol = ccol_up
        n = n_up
        Np = Mp

    x = _compose_final(feats3, ups[3], locs[3], locs[2], locs[1], locs[0])

    y2 = _head_call(x, params['pW0'], params['pb0'], params['pW1'],
                    params['pb1'], params['pW2'], params['pb2'])
    mh, rh = _stats(y2[:n0])
    w_clu16 = jnp.zeros((64, 16), jnp.float32).at[:, :13].set(params['W_clu'])
    out = _final_call(y2, params['bn2_g'], mh, rh, params['bn2_b'], w_clu16)
    return out[:n0, :13]
